# baseline (speedup 1.0000x reference)
import functools

import jax
import jax.numpy as jnp
from jax.experimental import pallas as pl
from jax.experimental.pallas import tpu as pltpu

_C = 21
_NOBJ = 8


def _match_body(boxes_ref, vals_ref, loc_ref, cls_ref, priors_ref,
                partials_ref, ceneg_ref):
    P = priors_ref.shape[1]
    pr = priors_ref[...]
    pcx, pcy, pw, ph = pr[0:1], pr[1:2], pr[2:3], pr[3:4]
    px0 = pcx - pw / 2.
    py0 = pcy - ph / 2.
    px1 = pcx + pw / 2.
    py1 = pcy + ph / 2.
    area_p = (px1 - px0) * (py1 - py0)

    b = boxes_ref[0]
    bx0, by0, bx1, by1 = b[:, 0:1], b[:, 1:2], b[:, 2:3], b[:, 3:4]
    area_b = (bx1 - bx0) * (by1 - by0)

    lt_x = jnp.maximum(bx0, px0)
    lt_y = jnp.maximum(by0, py0)
    rb_x = jnp.minimum(bx1, px1)
    rb_y = jnp.minimum(by1, py1)
    iw = jnp.maximum(rb_x - lt_x, 0.)
    ih = jnp.maximum(rb_y - lt_y, 0.)
    inter = iw * ih
    iou = inter / (area_b + area_p - inter)

    lane = jax.lax.broadcasted_iota(jnp.int32, (_NOBJ, P), 1)
    row_max = jnp.max(iou, axis=1, keepdims=True)
    best_p = jnp.min(jnp.where(iou == row_max, lane, P), axis=1,
                     keepdims=True)
    iou = jnp.where(lane == best_p, 1.0, iou)

    col_max = jnp.max(iou, axis=0, keepdims=True)
    pos = col_max >= 0.5
    oid = jax.lax.broadcasted_iota(jnp.int32, (_NOBJ, P), 0)
    sel = jnp.min(jnp.where(iou == col_max, oid, _NOBJ), axis=0,
                  keepdims=True)
    onehot = (oid == sel).astype(jnp.float32)

    g = jax.lax.dot_general(vals_ref[0], onehot, (((1,), (0,)), ((), ())),
                            precision=jax.lax.Precision.HIGHEST,
                            preferred_element_type=jnp.float32)
    gx0, gy0, gx1, gy1 = g[0:1], g[1:2], g[2:3], g[3:4]
    tc = (g[4:5] + 0.5).astype(jnp.int32)
    tc = jnp.where(pos, tc, _C - 1)

    gcx = (gx0 + gx1) / 2.
    gcy = (gy0 + gy1) / 2.
    gw = gx1 - gx0
    gh = gy1 - gy0
    t0 = (gcx - pcx) / (pw / 10.)
    t1 = (gcy - pcy) / (ph / 10.)
    t2 = jnp.log(gw / pw) * 5.
    t3 = jnp.log(gh / ph) * 5.
    tgt = jnp.concatenate([t0, t1, t2, t3], axis=0)

    d = loc_ref[0] - tgt
    ad = jnp.abs(d)
    sl1 = jnp.where(ad < 1.0, 0.5 * d * d, ad - 0.5)
    posf = pos.astype(jnp.float32)
    loc_sum = jnp.sum(sl1 * posf)

    cls = cls_ref[0]
    m = jnp.max(cls, axis=0, keepdims=True)
    cid = jax.lax.broadcasted_iota(jnp.int32, (_C, P), 0)
    stacked = jnp.concatenate(
        [jnp.exp(cls - m), jnp.where(cid == tc, cls, 0.)], axis=0)
    kid = jax.lax.broadcasted_iota(jnp.int32, (2, 2 * _C), 1) // _C
    rid = jax.lax.broadcasted_iota(jnp.int32, (2, 2 * _C), 0)
    red = (kid == rid).astype(jnp.float32)
    sums = jax.lax.dot_general(
        red, stacked, (((1,), (0,)), ((), ())),
        precision=jax.lax.Precision.HIGHEST,
        preferred_element_type=jnp.float32)
    ce = m + jnp.log(sums[0:1]) - sums[1:2]

    conf_pos = jnp.sum(ce * posf)
    n_pos = jnp.sum(posf)
    ceneg_ref[0] = jnp.maximum(jnp.where(pos, 0., ce), 0.)
    partials_ref[0] = jnp.concatenate(
        [jnp.full((1, 1), loc_sum), jnp.full((1, 1), conf_pos),
         jnp.full((1, 1), n_pos), jnp.zeros((1, 1))], axis=1)


def _mine_body(ceneg_ref, partials_ref, out_ref):
    B, P = ceneg_ref.shape
    v = ceneg_ref[...]
    bits = jax.lax.bitcast_convert_type(v, jnp.int32)
    p = partials_ref[...]
    n_pos = p[:, 2:3]
    k = jnp.minimum(3 * n_pos.astype(jnp.int32), P)

    def step(_, carry):
        lo, hi = carry
        mid = lo + (hi - lo) // 2
        cnt = jnp.sum((bits >= mid).astype(jnp.int32), axis=1, keepdims=True)
        ge = cnt >= k
        return (jnp.where(ge, mid, lo), jnp.where(ge, hi, mid))

    lo0 = jnp.zeros((B, 1), jnp.int32)
    hi0 = jnp.full((B, 1), 0x7f800000, jnp.int32)
    lo, _ = jax.lax.fori_loop(0, 31, step, (lo0, hi0))
    thr = jax.lax.bitcast_convert_type(lo, jnp.float32)
    gt = bits > lo
    cnt_gt = jnp.sum(gt.astype(jnp.int32), axis=1, keepdims=True)
    sum_gt = jnp.sum(jnp.where(gt, v, 0.), axis=1, keepdims=True)
    hard = sum_gt + (k - cnt_gt).astype(jnp.float32) * thr

    n_pos_sum = jnp.sum(n_pos)
    conf_loss = (jnp.sum(hard) + jnp.sum(p[:, 1:2])) / n_pos_sum
    loc_loss = jnp.sum(p[:, 0:1]) / n_pos_sum
    total = conf_loss + loc_loss
    out_ref[...] = jnp.concatenate(
        [jnp.full((1, 1), total), jnp.full((1, 1), loc_loss),
         jnp.full((1, 1), conf_loss), jnp.zeros((1, 1))], axis=1)


@jax.jit
def kernel(pred_loc, pred_cls, b_boxes, b_labels, priors_cxcy):
    B, P, C = pred_cls.shape
    loc_t = jnp.transpose(pred_loc, (0, 2, 1))
    cls_t = jnp.transpose(pred_cls, (0, 2, 1))
    priors_t = jnp.transpose(priors_cxcy, (1, 0))
    vals = jnp.concatenate(
        [jnp.transpose(b_boxes, (0, 2, 1)),
         b_labels.astype(jnp.float32)[:, None, :]], axis=1)

    partials, ceneg = pl.pallas_call(
        _match_body,
        grid=(B,),
        in_specs=[
            pl.BlockSpec((1, _NOBJ, 4), lambda i: (i, 0, 0)),
            pl.BlockSpec((1, 5, _NOBJ), lambda i: (i, 0, 0)),
            pl.BlockSpec((1, 4, P), lambda i: (i, 0, 0)),
            pl.BlockSpec((1, C, P), lambda i: (i, 0, 0)),
            pl.BlockSpec((4, P), lambda i: (0, 0)),
        ],
        out_specs=[
            pl.BlockSpec((1, 1, 4), lambda i: (i, 0, 0)),
            pl.BlockSpec((1, 1, P), lambda i: (i, 0, 0)),
        ],
        out_shape=[
            jax.ShapeDtypeStruct((B, 1, 4), jnp.float32),
            jax.ShapeDtypeStruct((B, 1, P), jnp.float32),
        ],
    )(b_boxes, vals, loc_t, cls_t, priors_t)

    out = pl.pallas_call(
        _mine_body,
        in_specs=[
            pl.BlockSpec((B, P), lambda: (0, 0)),
            pl.BlockSpec((B, 4), lambda: (0, 0)),
        ],
        out_specs=pl.BlockSpec((1, 4), lambda: (0, 0)),
        out_shape=jax.ShapeDtypeStruct((1, 4), jnp.float32),
    )(ceneg.reshape(B, P), partials.reshape(B, 4))

    return (out[0, 0], out[0, 1], out[0, 2])


# speedup vs baseline: 1.1934x; 1.1934x over previous
import jax
import jax.numpy as jnp
from jax.experimental import pallas as pl
from jax.experimental.pallas import tpu as pltpu

_C = 21
_NOBJ = 8


def _match_body(boxes_ref, vals_ref, loc_ref, priors_ref,
                partials_ref, pos_ref, tc_ref):
    P = priors_ref.shape[1]
    pr = priors_ref[...]
    pcx, pcy, pw, ph = pr[0:1], pr[1:2], pr[2:3], pr[3:4]
    px0 = pcx - pw / 2.
    py0 = pcy - ph / 2.
    px1 = pcx + pw / 2.
    py1 = pcy + ph / 2.
    area_p = (px1 - px0) * (py1 - py0)

    b = boxes_ref[0]
    bx0, by0, bx1, by1 = b[:, 0:1], b[:, 1:2], b[:, 2:3], b[:, 3:4]
    area_b = (bx1 - bx0) * (by1 - by0)

    lt_x = jnp.maximum(bx0, px0)
    lt_y = jnp.maximum(by0, py0)
    rb_x = jnp.minimum(bx1, px1)
    rb_y = jnp.minimum(by1, py1)
    iw = jnp.maximum(rb_x - lt_x, 0.)
    ih = jnp.maximum(rb_y - lt_y, 0.)
    inter = iw * ih
    iou = inter / (area_b + area_p - inter)

    lane = jax.lax.broadcasted_iota(jnp.int32, (_NOBJ, P), 1)
    row_max = jnp.max(iou, axis=1, keepdims=True)
    best_p = jnp.min(jnp.where(iou == row_max, lane, P), axis=1,
                     keepdims=True)
    iou = jnp.where(lane == best_p, 1.0, iou)

    col_max = jnp.max(iou, axis=0, keepdims=True)
    pos = col_max >= 0.5
    oid = jax.lax.broadcasted_iota(jnp.int32, (_NOBJ, P), 0)
    sel = jnp.min(jnp.where(iou == col_max, oid, _NOBJ), axis=0,
                  keepdims=True)
    onehot = (oid == sel).astype(jnp.float32)

    g = jax.lax.dot_general(vals_ref[0], onehot, (((1,), (0,)), ((), ())),
                            precision=jax.lax.Precision.HIGHEST,
                            preferred_element_type=jnp.float32)
    gx0, gy0, gx1, gy1 = g[0:1], g[1:2], g[2:3], g[3:4]
    tc = (g[4:5] + 0.5).astype(jnp.int32)
    tc = jnp.where(pos, tc, _C - 1)

    gcx = (gx0 + gx1) / 2.
    gcy = (gy0 + gy1) / 2.
    gw = gx1 - gx0
    gh = gy1 - gy0
    t0 = (gcx - pcx) / (pw / 10.)
    t1 = (gcy - pcy) / (ph / 10.)
    t2 = jnp.log(gw / pw) * 5.
    t3 = jnp.log(gh / ph) * 5.
    tgt = jnp.concatenate([t0, t1, t2, t3], axis=0)

    d = loc_ref[0] - tgt
    ad = jnp.abs(d)
    sl1 = jnp.where(ad < 1.0, 0.5 * d * d, ad - 0.5)
    posf = pos.astype(jnp.float32)
    loc_sum = jnp.sum(sl1 * posf)
    n_pos = jnp.sum(posf)

    pos_ref[0] = posf
    tc_ref[0] = tc
    partials_ref[0] = jnp.concatenate(
        [jnp.full((1, 1), loc_sum), jnp.full((1, 1), n_pos),
         jnp.zeros((1, 2))], axis=1)


def _ce_body(cls_ref, tc_ref, pos_ref, partials_ref, ceneg_ref):
    cls = cls_ref[0]
    tc = tc_ref[0]
    posf = pos_ref[0]
    P = cls.shape[1]
    m = jnp.max(cls, axis=0, keepdims=True)
    lse = jnp.log(jnp.sum(jnp.exp(cls - m), axis=0, keepdims=True))
    cid = jax.lax.broadcasted_iota(jnp.int32, (_C, P), 0)
    logit_tc = jnp.sum(jnp.where(cid == tc, cls, 0.), axis=0, keepdims=True)
    ce = m + lse - logit_tc

    conf_pos = jnp.sum(ce * posf)
    ceneg_ref[0] = jnp.maximum(jnp.where(posf > 0.5, 0., ce), 0.)
    partials_ref[0] = jnp.concatenate(
        [jnp.full((1, 1), conf_pos), jnp.zeros((1, 3))], axis=1)


def _mine_body(ceneg_ref, pm_ref, pc_ref, out_ref):
    B, P = ceneg_ref.shape
    v = ceneg_ref[...]
    bits = jax.lax.bitcast_convert_type(v, jnp.int32)
    pm = pm_ref[...]
    pc = pc_ref[...]
    n_pos = pm[:, 1:2]
    k = jnp.minimum(3 * n_pos.astype(jnp.int32), P)

    def step(_, carry):
        lo, hi = carry
        mid = lo + (hi - lo) // 2
        cnt = jnp.sum((bits >= mid).astype(jnp.int32), axis=1, keepdims=True)
        ge = cnt >= k
        return (jnp.where(ge, mid, lo), jnp.where(ge, hi, mid))

    lo0 = jnp.zeros((B, 1), jnp.int32)
    hi0 = jnp.full((B, 1), 0x7f800000, jnp.int32)
    lo, _ = jax.lax.fori_loop(0, 31, step, (lo0, hi0))
    thr = jax.lax.bitcast_convert_type(lo, jnp.float32)
    gt = bits > lo
    cnt_gt = jnp.sum(gt.astype(jnp.int32), axis=1, keepdims=True)
    sum_gt = jnp.sum(jnp.where(gt, v, 0.), axis=1, keepdims=True)
    hard = sum_gt + (k - cnt_gt).astype(jnp.float32) * thr

    n_pos_sum = jnp.sum(n_pos)
    conf_loss = (jnp.sum(hard) + jnp.sum(pc[:, 0:1])) / n_pos_sum
    loc_loss = jnp.sum(pm[:, 0:1]) / n_pos_sum
    total = conf_loss + loc_loss
    out_ref[...] = jnp.concatenate(
        [jnp.full((1, 1), total), jnp.full((1, 1), loc_loss),
         jnp.full((1, 1), conf_loss), jnp.zeros((1, 1))], axis=1)


@jax.jit
def kernel(pred_loc, pred_cls, b_boxes, b_labels, priors_cxcy):
    B, P, C = pred_cls.shape
    loc_t = jnp.transpose(pred_loc, (0, 2, 1))
    cls_t = jnp.transpose(pred_cls, (0, 2, 1))
    priors_t = jnp.transpose(priors_cxcy, (1, 0))
    vals = jnp.concatenate(
        [jnp.transpose(b_boxes, (0, 2, 1)),
         b_labels.astype(jnp.float32)[:, None, :]], axis=1)

    pm, pos, tc = pl.pallas_call(
        _match_body,
        grid=(B,),
        in_specs=[
            pl.BlockSpec((1, _NOBJ, 4), lambda i: (i, 0, 0)),
            pl.BlockSpec((1, 5, _NOBJ), lambda i: (i, 0, 0)),
            pl.BlockSpec((1, 4, P), lambda i: (i, 0, 0)),
            pl.BlockSpec((4, P), lambda i: (0, 0)),
        ],
        out_specs=[
            pl.BlockSpec((1, 1, 4), lambda i: (i, 0, 0)),
            pl.BlockSpec((1, 1, P), lambda i: (i, 0, 0)),
            pl.BlockSpec((1, 1, P), lambda i: (i, 0, 0)),
        ],
        out_shape=[
            jax.ShapeDtypeStruct((B, 1, 4), jnp.float32),
            jax.ShapeDtypeStruct((B, 1, P), jnp.float32),
            jax.ShapeDtypeStruct((B, 1, P), jnp.int32),
        ],
    )(b_boxes, vals, loc_t, priors_t)

    pc, ceneg = pl.pallas_call(
        _ce_body,
        grid=(B,),
        in_specs=[
            pl.BlockSpec((1, C, P), lambda i: (i, 0, 0)),
            pl.BlockSpec((1, 1, P), lambda i: (i, 0, 0)),
            pl.BlockSpec((1, 1, P), lambda i: (i, 0, 0)),
        ],
        out_specs=[
            pl.BlockSpec((1, 1, 4), lambda i: (i, 0, 0)),
            pl.BlockSpec((1, 1, P), lambda i: (i, 0, 0)),
        ],
        out_shape=[
            jax.ShapeDtypeStruct((B, 1, 4), jnp.float32),
            jax.ShapeDtypeStruct((B, 1, P), jnp.float32),
        ],
    )(cls_t, tc, pos)

    out = pl.pallas_call(
        _mine_body,
        in_specs=[
            pl.BlockSpec((B, P), lambda: (0, 0)),
            pl.BlockSpec((B, 4), lambda: (0, 0)),
            pl.BlockSpec((B, 4), lambda: (0, 0)),
        ],
        out_specs=pl.BlockSpec((1, 4), lambda: (0, 0)),
        out_shape=jax.ShapeDtypeStruct((1, 4), jnp.float32),
    )(ceneg.reshape(B, P), pm.reshape(B, 4), pc.reshape(B, 4))

    return (out[0, 0], out[0, 1], out[0, 2])


# speedup vs baseline: 1.3240x; 1.1095x over previous
import jax
import jax.numpy as jnp
from jax.experimental import pallas as pl
from jax.experimental.pallas import tpu as pltpu

_C = 21
_NOBJ = 8


def _match_body(boxes_ref, vals_ref, priors_ref, pos_ref, tc_ref, tgt_ref):
    P = priors_ref.shape[1]
    pr = priors_ref[...]
    pcx, pcy, pw, ph = pr[0:1], pr[1:2], pr[2:3], pr[3:4]
    px0 = pcx - pw / 2.
    py0 = pcy - ph / 2.
    px1 = pcx + pw / 2.
    py1 = pcy + ph / 2.
    area_p = (px1 - px0) * (py1 - py0)

    b = boxes_ref[0]
    bx0, by0, bx1, by1 = b[:, 0:1], b[:, 1:2], b[:, 2:3], b[:, 3:4]
    area_b = (bx1 - bx0) * (by1 - by0)

    lt_x = jnp.maximum(bx0, px0)
    lt_y = jnp.maximum(by0, py0)
    rb_x = jnp.minimum(bx1, px1)
    rb_y = jnp.minimum(by1, py1)
    iw = jnp.maximum(rb_x - lt_x, 0.)
    ih = jnp.maximum(rb_y - lt_y, 0.)
    inter = iw * ih
    iou = inter / (area_b + area_p - inter)

    lane = jax.lax.broadcasted_iota(jnp.int32, (_NOBJ, P), 1)
    row_max = jnp.max(iou, axis=1, keepdims=True)
    best_p = jnp.min(jnp.where(iou == row_max, lane, P), axis=1,
                     keepdims=True)
    iou = jnp.where(lane == best_p, 1.0, iou)

    col_max = jnp.max(iou, axis=0, keepdims=True)
    pos = col_max >= 0.5
    oid = jax.lax.broadcasted_iota(jnp.int32, (_NOBJ, P), 0)
    sel = jnp.min(jnp.where(iou == col_max, oid, _NOBJ), axis=0,
                  keepdims=True)
    onehot = (oid == sel).astype(jnp.float32)

    g = jax.lax.dot_general(vals_ref[0], onehot, (((1,), (0,)), ((), ())),
                            precision=jax.lax.Precision.HIGHEST,
                            preferred_element_type=jnp.float32)
    gx0, gy0, gx1, gy1 = g[0:1], g[1:2], g[2:3], g[3:4]
    tc = (g[4:5] + 0.5).astype(jnp.int32)
    tc = jnp.where(pos, tc, _C - 1)

    gcx = (gx0 + gx1) / 2.
    gcy = (gy0 + gy1) / 2.
    gw = gx1 - gx0
    gh = gy1 - gy0
    t0 = (gcx - pcx) / (pw / 10.)
    t1 = (gcy - pcy) / (ph / 10.)
    t2 = jnp.log(gw / pw) * 5.
    t3 = jnp.log(gh / ph) * 5.

    pos_ref[0] = pos.astype(jnp.float32)
    tc_ref[0] = tc
    tgt_ref[0] = jnp.concatenate([t0, t1, t2, t3], axis=0)


def _ce_body(cls_ref, loc_ref, tgt_ref, tc_ref, pos_ref,
             out_ref, ceneg_ref, row_ref):
    i = pl.program_id(0)
    B = pl.num_programs(0)
    cls = cls_ref[0]
    tc = tc_ref[0]
    posf = pos_ref[0]
    P = cls.shape[1]

    d = loc_ref[0] - tgt_ref[0]
    ad = jnp.abs(d)
    sl1 = jnp.where(ad < 1.0, 0.5 * d * d, ad - 0.5)
    row_ref[i, 0] = jnp.sum(sl1 * posf)
    row_ref[i, 2] = jnp.sum(posf)

    m = jnp.max(cls, axis=0, keepdims=True)
    lse = jnp.log(jnp.sum(jnp.exp(cls - m), axis=0, keepdims=True))
    cid = jax.lax.broadcasted_iota(jnp.int32, (_C, P), 0)
    logit_tc = jnp.sum(jnp.where(cid == tc, cls, 0.), axis=0, keepdims=True)
    ce = m + lse - logit_tc
    row_ref[i, 1] = jnp.sum(ce * posf)
    ceneg_ref[pl.ds(i, 1), :] = jnp.maximum(jnp.where(posf > 0.5, 0., ce), 0.)

    @pl.when(i == B - 1)
    def _mine():
        v = ceneg_ref[...]
        bits = jax.lax.bitcast_convert_type(v, jnp.int32)
        rid = jax.lax.broadcasted_iota(jnp.int32, (B, 1), 0)
        loc_sum = jnp.zeros((B, 1), jnp.float32)
        conf_pos = jnp.zeros((B, 1), jnp.float32)
        n_pos = jnp.zeros((B, 1), jnp.float32)
        for r in range(B):
            sel_r = rid == r
            loc_sum = jnp.where(sel_r, row_ref[r, 0], loc_sum)
            conf_pos = jnp.where(sel_r, row_ref[r, 1], conf_pos)
            n_pos = jnp.where(sel_r, row_ref[r, 2], n_pos)
        k = jnp.minimum(3 * n_pos.astype(jnp.int32), P)

        def step(_, carry):
            lo, hi = carry
            mid = lo + (hi - lo) // 2
            cnt = jnp.sum((bits >= mid).astype(jnp.int32), axis=1,
                          keepdims=True)
            ge = cnt >= k
            return (jnp.where(ge, mid, lo), jnp.where(ge, hi, mid))

        lo0 = jnp.zeros((B, 1), jnp.int32)
        hi0 = jnp.full((B, 1), 0x7f800000, jnp.int32)
        lo, _ = jax.lax.fori_loop(0, 31, step, (lo0, hi0))
        thr = jax.lax.bitcast_convert_type(lo, jnp.float32)
        gt = bits > lo
        cnt_gt = jnp.sum(gt.astype(jnp.int32), axis=1, keepdims=True)
        sum_gt = jnp.sum(jnp.where(gt, v, 0.), axis=1, keepdims=True)
        hard = sum_gt + (k - cnt_gt).astype(jnp.float32) * thr

        n_pos_sum = jnp.sum(n_pos)
        conf_loss = (jnp.sum(hard) + jnp.sum(conf_pos)) / n_pos_sum
        loc_loss = jnp.sum(loc_sum) / n_pos_sum
        total = conf_loss + loc_loss
        out_ref[...] = jnp.concatenate(
            [jnp.full((1, 1), total), jnp.full((1, 1), loc_loss),
             jnp.full((1, 1), conf_loss), jnp.zeros((1, 1))], axis=1)


@jax.jit
def kernel(pred_loc, pred_cls, b_boxes, b_labels, priors_cxcy):
    B, P, C = pred_cls.shape
    loc_t = jnp.transpose(pred_loc, (0, 2, 1))
    cls_t = jnp.transpose(pred_cls, (0, 2, 1))
    priors_t = jnp.transpose(priors_cxcy, (1, 0))
    vals = jnp.concatenate(
        [jnp.transpose(b_boxes, (0, 2, 1)),
         b_labels.astype(jnp.float32)[:, None, :]], axis=1)

    pos, tc, tgt = pl.pallas_call(
        _match_body,
        grid=(B,),
        in_specs=[
            pl.BlockSpec((1, _NOBJ, 4), lambda i: (i, 0, 0)),
            pl.BlockSpec((1, 5, _NOBJ), lambda i: (i, 0, 0)),
            pl.BlockSpec((4, P), lambda i: (0, 0)),
        ],
        out_specs=[
            pl.BlockSpec((1, 1, P), lambda i: (i, 0, 0)),
            pl.BlockSpec((1, 1, P), lambda i: (i, 0, 0)),
            pl.BlockSpec((1, 4, P), lambda i: (i, 0, 0)),
        ],
        out_shape=[
            jax.ShapeDtypeStruct((B, 1, P), jnp.float32),
            jax.ShapeDtypeStruct((B, 1, P), jnp.int32),
            jax.ShapeDtypeStruct((B, 4, P), jnp.float32),
        ],
    )(b_boxes, vals, priors_t)

    out = pl.pallas_call(
        _ce_body,
        grid=(B,),
        in_specs=[
            pl.BlockSpec((1, C, P), lambda i: (i, 0, 0)),
            pl.BlockSpec((1, 4, P), lambda i: (i, 0, 0)),
            pl.BlockSpec((1, 4, P), lambda i: (i, 0, 0)),
            pl.BlockSpec((1, 1, P), lambda i: (i, 0, 0)),
            pl.BlockSpec((1, 1, P), lambda i: (i, 0, 0)),
        ],
        out_specs=pl.BlockSpec((1, 4), lambda i: (0, 0)),
        out_shape=jax.ShapeDtypeStruct((1, 4), jnp.float32),
        scratch_shapes=[
            pltpu.VMEM((B, P), jnp.float32),
            pltpu.SMEM((B, 4), jnp.float32),
        ],
    )(cls_t, loc_t, tgt, tc, pos)

    return (out[0, 0], out[0, 1], out[0, 2])
